# baseline (speedup 1.0000x reference)
import functools

import jax
import jax.numpy as jnp
from jax import lax
from jax.experimental import pallas as pl
from jax.experimental.pallas import tpu as pltpu
from jax.experimental.pallas import tpu_sc as plsc

N = 20000
C = 81
BD = 324
K = 128
BLK = 400
NBLK = N // BLK

SC_CORES = 2
SC_SUBCORES = 16
ZROWS = 200
ZCHUNKS = N // ZROWS
ZITER = -(-ZCHUNKS // SC_SUBCORES)


def _zero_body(iw_out, ow_out, zbuf):
    c = lax.axis_index("c")
    s = lax.axis_index("s")
    zero16 = jnp.zeros((16,), jnp.float32)

    def zrow(i, carry):
        for j in range(BD // 16):
            zbuf[i, pl.ds(j * 16, 16)] = zero16
        zbuf[i, pl.ds(BD - 16, 16)] = zero16
        return carry

    lax.fori_loop(0, ZROWS, zrow, 0)

    for k in range(ZITER):
        chunk = s + SC_SUBCORES * k

        @pl.when(jnp.logical_and(chunk < ZCHUNKS, c == 0))
        def _():
            pltpu.sync_copy(zbuf, iw_out.at[pl.ds(chunk * ZROWS, ZROWS), :])

        @pl.when(jnp.logical_and(chunk < ZCHUNKS, c == 1))
        def _():
            pltpu.sync_copy(zbuf, ow_out.at[pl.ds(chunk * ZROWS, ZROWS), :])


_zero_call = functools.partial(
    pl.kernel,
    mesh=plsc.VectorSubcoreMesh(core_axis_name="c", subcore_axis_name="s"),
    out_type=[
        jax.ShapeDtypeStruct((N, BD), jnp.float32),
        jax.ShapeDtypeStruct((N, BD), jnp.float32),
    ],
    scratch_types=[pltpu.VMEM((ZROWS, BD), jnp.float32)],
)(_zero_body)


def _loss_kernel(cls_ref, lab_ref, bp_ref, bt_ref, iw_ref, ow_ref, out_ref):
    x = cls_ref[...]
    lab = lab_ref[...]
    m = jnp.max(x, axis=1, keepdims=True)
    e = jnp.exp(x - m)
    s = jnp.sum(e, axis=1, keepdims=True)
    col = lax.broadcasted_iota(jnp.int32, (BLK, C), 1)
    xl = jnp.sum(jnp.where(col == lab, x, 0.0), axis=1, keepdims=True)
    loss_cls = jnp.log(s) + m - xl

    d = iw_ref[...] * (bp_ref[...] - bt_ref[...])
    a = jnp.abs(d)
    sign = (a < 1.0).astype(jnp.float32)
    in_loss = sign * 0.5 * d * d + (1.0 - sign) * (a - 0.5)
    loss_bbox = jnp.sum(ow_ref[...] * in_loss, axis=1, keepdims=True)

    out_ref[...] = loss_cls + loss_bbox


NG = N // 8


def _select_kernel(loss_ref, lab_ref, labout_ref, gidx_ref, sbits_ref,
                   v_ref, keep_ref):
    gid = (lax.broadcasted_iota(jnp.int32, (NBLK, BLK), 0) * BLK
           + lax.broadcasted_iota(jnp.int32, (NBLK, BLK), 1))
    big = jnp.int32(2**30)
    v_ref[...] = loss_ref[...]
    keep_ref[...] = jnp.zeros((NBLK, BLK), jnp.float32)
    gidx_ref[...] = jnp.zeros((K, 1), jnp.int32)
    subl = lax.broadcasted_iota(jnp.int32, (K, 1), 0)

    def body(i, c):
        v = v_ref[...]
        m = jnp.max(v)
        cand = jnp.where(v == m, gid, big)
        fi = jnp.min(cand)
        hit = gid == fi
        keep_ref[...] = jnp.where(hit, 1.0, keep_ref[...])
        v_ref[...] = jnp.where(hit, -jnp.inf, v)
        gidx_ref[...] = jnp.where(subl == i, fi // 8, gidx_ref[...])
        return c

    lax.fori_loop(0, K, body, 0)
    keep = keep_ref[...]
    labout_ref[...] = jnp.where(keep > 0.0, lab_ref[...], -1)

    lane2 = lax.broadcasted_iota(jnp.int32, (NBLK, BLK), 1)
    w = lax.shift_left(jnp.int32(1), lane2 % 8).astype(jnp.float32)
    gsel = (lax.broadcasted_iota(jnp.int32, (BLK, BLK // 8), 0) // 8
            == lax.broadcasted_iota(jnp.int32, (BLK, BLK // 8), 1))
    bits = jnp.dot(keep * w, gsel.astype(jnp.float32),
                   preferred_element_type=jnp.float32)

    g = gidx_ref[...]
    rsel = (lax.broadcasted_iota(jnp.int32, (K, NBLK), 1)
            == g // (BLK // 8)).astype(jnp.float32)
    p1 = jnp.dot(rsel, bits, preferred_element_type=jnp.float32)
    csel = (lax.broadcasted_iota(jnp.int32, (K, BLK // 8), 1)
            == g % (BLK // 8)).astype(jnp.float32)
    sbits_ref[...] = jnp.sum(p1 * csel, axis=1,
                             keepdims=True).astype(jnp.int32)


def _scatter_kernel(g_ref, sbits_ref, iw_ref, ow_ref, iwz_ref, owz_ref,
                    iwo_ref, owo_ref, bufi, bufo, semg, semo):
    def fire(i, c):
        g = g_ref[i]
        pltpu.make_async_copy(
            iw_ref.at[pl.ds(g * 8, 8), :], bufi.at[i], semg).start()
        pltpu.make_async_copy(
            ow_ref.at[pl.ds(g * 8, 8), :], bufo.at[i], semg).start()
        return c

    lax.fori_loop(0, K, fire, 0)

    def drain_in(i, c):
        g = g_ref[i]
        pltpu.make_async_copy(
            iw_ref.at[pl.ds(g * 8, 8), :], bufi.at[i], semg).wait()
        pltpu.make_async_copy(
            ow_ref.at[pl.ds(g * 8, 8), :], bufo.at[i], semg).wait()
        return c

    lax.fori_loop(0, K, drain_in, 0)

    sb3 = sbits_ref[...].reshape(K, 1, 1)
    sub3 = lax.broadcasted_iota(jnp.int32, (1, 8, 1), 1)
    rowm = (lax.shift_right_logical(sb3, sub3) & 1) > 0
    bufi[...] = jnp.where(rowm, bufi[...], 0.0)
    bufo[...] = jnp.where(rowm, bufo[...], 0.0)

    def fire_out(i, c):
        g = g_ref[i]
        pltpu.make_async_copy(
            bufi.at[i], iwo_ref.at[pl.ds(g * 8, 8), :], semo).start()
        pltpu.make_async_copy(
            bufo.at[i], owo_ref.at[pl.ds(g * 8, 8), :], semo).start()
        return c

    lax.fori_loop(0, K, fire_out, 0)

    def drain_out(i, c):
        g = g_ref[i]
        pltpu.make_async_copy(
            bufi.at[i], iwo_ref.at[pl.ds(g * 8, 8), :], semo).wait()
        pltpu.make_async_copy(
            bufo.at[i], owo_ref.at[pl.ds(g * 8, 8), :], semo).wait()
        return c

    lax.fori_loop(0, K, drain_out, 0)


@jax.jit
def kernel(cls_score, bbox_pred, labels, bbox_targets, inside_ws, outside_ws):
    iw_z, ow_z = _zero_call()

    lab2 = labels.reshape(N, 1)
    loss = pl.pallas_call(
        _loss_kernel,
        grid=(NBLK,),
        in_specs=[
            pl.BlockSpec((BLK, C), lambda i: (i, 0)),
            pl.BlockSpec((BLK, 1), lambda i: (i, 0)),
            pl.BlockSpec((BLK, BD), lambda i: (i, 0)),
            pl.BlockSpec((BLK, BD), lambda i: (i, 0)),
            pl.BlockSpec((BLK, BD), lambda i: (i, 0)),
            pl.BlockSpec((BLK, BD), lambda i: (i, 0)),
        ],
        out_specs=pl.BlockSpec((BLK, 1), lambda i: (i, 0)),
        out_shape=jax.ShapeDtypeStruct((N, 1), jnp.float32),
    )(cls_score, lab2, bbox_pred, bbox_targets, inside_ws, outside_ws)

    lab_ohem2d, gidxv, sbitsv = pl.pallas_call(
        _select_kernel,
        in_specs=[
            pl.BlockSpec((NBLK, BLK), lambda: (0, 0)),
            pl.BlockSpec((NBLK, BLK), lambda: (0, 0)),
        ],
        out_specs=[
            pl.BlockSpec((NBLK, BLK), lambda: (0, 0)),
            pl.BlockSpec((K, 1), lambda: (0, 0)),
            pl.BlockSpec((K, 1), lambda: (0, 0)),
        ],
        out_shape=[
            jax.ShapeDtypeStruct((NBLK, BLK), labels.dtype),
            jax.ShapeDtypeStruct((K, 1), jnp.int32),
            jax.ShapeDtypeStruct((K, 1), jnp.int32),
        ],
        scratch_shapes=[
            pltpu.VMEM((NBLK, BLK), jnp.float32),
            pltpu.VMEM((NBLK, BLK), jnp.float32),
        ],
    )(loss.reshape(NBLK, BLK), labels.reshape(NBLK, BLK))

    grid_spec = pltpu.PrefetchScalarGridSpec(
        num_scalar_prefetch=1,
        grid=(1,),
        in_specs=[
            pl.BlockSpec((K, 1), lambda i, g: (0, 0)),
            pl.BlockSpec(memory_space=pl.ANY),
            pl.BlockSpec(memory_space=pl.ANY),
            pl.BlockSpec(memory_space=pl.ANY),
            pl.BlockSpec(memory_space=pl.ANY),
        ],
        out_specs=[
            pl.BlockSpec(memory_space=pl.ANY),
            pl.BlockSpec(memory_space=pl.ANY),
        ],
        scratch_shapes=[
            pltpu.VMEM((K, 8, BD), jnp.float32),
            pltpu.VMEM((K, 8, BD), jnp.float32),
            pltpu.SemaphoreType.DMA,
            pltpu.SemaphoreType.DMA,
        ],
    )
    iw_o, ow_o = pl.pallas_call(
        _scatter_kernel,
        grid_spec=grid_spec,
        out_shape=[
            jax.ShapeDtypeStruct((N, BD), jnp.float32),
            jax.ShapeDtypeStruct((N, BD), jnp.float32),
        ],
        input_output_aliases={4: 0, 5: 1},
    )(gidxv.reshape(K), sbitsv,
      inside_ws, outside_ws, iw_z, ow_z)

    return (lab_ohem2d.reshape(N), iw_o, ow_o)


# speedup vs baseline: 1.1299x; 1.1299x over previous
import functools

import jax
import jax.numpy as jnp
from jax import lax
from jax.experimental import pallas as pl
from jax.experimental.pallas import tpu as pltpu
from jax.experimental.pallas import tpu_sc as plsc

N = 20000
C = 81
BD = 324
K = 128
BLK = 400
NBLK = N // BLK

SC_CORES = 2
SC_SUBCORES = 16
ZROWS = 200
ZCHUNKS = N // ZROWS
ZITER = -(-ZCHUNKS // SC_SUBCORES)


def _zero_body(iw_out, ow_out, zbuf):
    c = lax.axis_index("c")
    s = lax.axis_index("s")
    zero16 = jnp.zeros((16,), jnp.float32)

    def zrow(i, carry):
        for j in range(BD // 16):
            zbuf[i, pl.ds(j * 16, 16)] = zero16
        zbuf[i, pl.ds(BD - 16, 16)] = zero16
        return carry

    lax.fori_loop(0, ZROWS, zrow, 0)

    for k in range(ZITER):
        chunk = s + SC_SUBCORES * k

        @pl.when(jnp.logical_and(chunk < ZCHUNKS, c == 0))
        def _():
            pltpu.sync_copy(zbuf, iw_out.at[pl.ds(chunk * ZROWS, ZROWS), :])

        @pl.when(jnp.logical_and(chunk < ZCHUNKS, c == 1))
        def _():
            pltpu.sync_copy(zbuf, ow_out.at[pl.ds(chunk * ZROWS, ZROWS), :])


_zero_call = functools.partial(
    pl.kernel,
    mesh=plsc.VectorSubcoreMesh(core_axis_name="c", subcore_axis_name="s"),
    out_type=[
        jax.ShapeDtypeStruct((N, BD), jnp.float32),
        jax.ShapeDtypeStruct((N, BD), jnp.float32),
    ],
    scratch_types=[pltpu.VMEM((ZROWS, BD), jnp.float32)],
)(_zero_body)


def _loss_kernel(cls_ref, lab_ref, bp_ref, bt_ref, iw_ref, ow_ref, out_ref):
    x = cls_ref[...]
    lab = lab_ref[...]
    m = jnp.max(x, axis=1, keepdims=True)
    e = jnp.exp(x - m)
    s = jnp.sum(e, axis=1, keepdims=True)
    col = lax.broadcasted_iota(jnp.int32, (BLK, C), 1)
    xl = jnp.sum(jnp.where(col == lab, x, 0.0), axis=1, keepdims=True)
    loss_cls = jnp.log(s) + m - xl

    d = iw_ref[...] * (bp_ref[...] - bt_ref[...])
    a = jnp.abs(d)
    sign = (a < 1.0).astype(jnp.float32)
    in_loss = sign * 0.5 * d * d + (1.0 - sign) * (a - 0.5)
    loss_bbox = jnp.sum(ow_ref[...] * in_loss, axis=1, keepdims=True)

    out_ref[...] = loss_cls + loss_bbox


NG = N // 8


def _select_kernel(loss_ref, lab_ref, labout_ref, gidx_ref, sbits_ref):
    v = loss_ref[...]
    u = lax.bitcast_convert_type(v, jnp.int32)
    ks = jnp.where(u >= 0, u, u ^ jnp.int32(0x7FFFFFFF))

    c0 = jnp.sum((ks >= 0).astype(jnp.float32))
    t0 = jnp.where(c0 >= K, jnp.int32(0), jnp.int32(-(2**31)))

    def tbody(bi, t_acc):
        t = t_acc | lax.shift_left(jnp.int32(1), 30 - bi)
        cnt = jnp.sum((ks >= t).astype(jnp.float32))
        return jnp.where(cnt >= K, t, t_acc)

    tthr = lax.fori_loop(0, 31, tbody, t0)

    gt = ks > tthr
    eq = ks == tthr
    need = jnp.float32(K) - jnp.sum(gt.astype(jnp.float32))

    eqf = eq.astype(jnp.float32)
    lt_b = (lax.broadcasted_iota(jnp.int32, (BLK, BLK), 0)
            <= lax.broadcasted_iota(jnp.int32, (BLK, BLK), 1)
            ).astype(jnp.float32)
    incl = jnp.dot(eqf, lt_b, preferred_element_type=jnp.float32)
    rowtot = jnp.sum(eqf, axis=1, keepdims=True)
    ls_r = (lax.broadcasted_iota(jnp.int32, (NBLK, NBLK), 1)
            < lax.broadcasted_iota(jnp.int32, (NBLK, NBLK), 0)
            ).astype(jnp.float32)
    rowpre = jnp.dot(ls_r, rowtot, preferred_element_type=jnp.float32)
    rank = rowpre + incl - eqf
    keep = jnp.where(jnp.logical_or(gt, jnp.logical_and(eq, rank < need)),
                     1.0, 0.0)

    labout_ref[...] = jnp.where(keep > 0.0, lab_ref[...], -1)

    NQ = BLK // 8
    lane2 = lax.broadcasted_iota(jnp.int32, (NBLK, BLK), 1)
    w = lax.shift_left(jnp.int32(1), lane2 % 8).astype(jnp.float32)
    gsel = (lax.broadcasted_iota(jnp.int32, (BLK, NQ), 0) // 8
            == lax.broadcasted_iota(jnp.int32, (BLK, NQ), 1))
    bits = jnp.dot(keep * w, gsel.astype(jnp.float32),
                   preferred_element_type=jnp.float32)

    nzf = (bits > 0.0).astype(jnp.float32)
    cellf = (lax.broadcasted_iota(jnp.int32, (NBLK, NQ), 0) * NQ
             + lax.broadcasted_iota(jnp.int32, (NBLK, NQ), 1)
             ).astype(jnp.float32)
    lt_q = (lax.broadcasted_iota(jnp.int32, (NQ, NQ), 0)
            <= lax.broadcasted_iota(jnp.int32, (NQ, NQ), 1)
            ).astype(jnp.float32)
    incl2 = jnp.dot(nzf, lt_q, preferred_element_type=jnp.float32)
    rowtot2 = jnp.sum(nzf, axis=1, keepdims=True)
    rowpre2 = jnp.dot(ls_r, rowtot2, preferred_element_type=jnp.float32)
    srank = rowpre2 + incl2 - nzf
    cnt2 = jnp.sum(nzf)

    s_iota = lax.broadcasted_iota(jnp.int32, (NBLK, NQ, K), 2
                                  ).astype(jnp.float32)
    oh = jnp.where(jnp.logical_and(srank[:, :, None] == s_iota,
                                   nzf[:, :, None] > 0.0), 1.0, 0.0)
    gidx_l = jnp.sum(oh * cellf[:, :, None], axis=(0, 1))
    sbits_l = jnp.sum(oh * bits[:, :, None], axis=(0, 1))

    g1 = jnp.sum(jnp.where(srank == 0.0, nzf * cellf, 0.0))
    b1 = jnp.sum(jnp.where(srank == 0.0, nzf * bits, 0.0))
    sl = lax.broadcasted_iota(jnp.int32, (K,), 0).astype(jnp.float32)
    gidx_l = jnp.where(sl < cnt2, gidx_l, g1)
    sbits_l = jnp.where(sl < cnt2, sbits_l, b1)

    gidx_ref[...] = gidx_l.astype(jnp.int32).reshape(K, 1)
    sbits_ref[...] = sbits_l.astype(jnp.int32).reshape(K, 1)


def _scatter_kernel(g_ref, sbits_ref, iw_ref, ow_ref, iwz_ref, owz_ref,
                    iwo_ref, owo_ref, bufi, bufo, semg, semo):
    def fire(i, c):
        g = g_ref[i]
        pltpu.make_async_copy(
            iw_ref.at[pl.ds(g * 8, 8), :], bufi.at[i], semg).start()
        pltpu.make_async_copy(
            ow_ref.at[pl.ds(g * 8, 8), :], bufo.at[i], semg).start()
        return c

    lax.fori_loop(0, K, fire, 0)

    def drain_in(i, c):
        g = g_ref[i]
        pltpu.make_async_copy(
            iw_ref.at[pl.ds(g * 8, 8), :], bufi.at[i], semg).wait()
        pltpu.make_async_copy(
            ow_ref.at[pl.ds(g * 8, 8), :], bufo.at[i], semg).wait()
        return c

    lax.fori_loop(0, K, drain_in, 0)

    sb3 = sbits_ref[...].reshape(K, 1, 1)
    sub3 = lax.broadcasted_iota(jnp.int32, (1, 8, 1), 1)
    rowm = (lax.shift_right_logical(sb3, sub3) & 1) > 0
    bufi[...] = jnp.where(rowm, bufi[...], 0.0)
    bufo[...] = jnp.where(rowm, bufo[...], 0.0)

    def fire_out(i, c):
        g = g_ref[i]
        pltpu.make_async_copy(
            bufi.at[i], iwo_ref.at[pl.ds(g * 8, 8), :], semo).start()
        pltpu.make_async_copy(
            bufo.at[i], owo_ref.at[pl.ds(g * 8, 8), :], semo).start()
        return c

    lax.fori_loop(0, K, fire_out, 0)

    def drain_out(i, c):
        g = g_ref[i]
        pltpu.make_async_copy(
            bufi.at[i], iwo_ref.at[pl.ds(g * 8, 8), :], semo).wait()
        pltpu.make_async_copy(
            bufo.at[i], owo_ref.at[pl.ds(g * 8, 8), :], semo).wait()
        return c

    lax.fori_loop(0, K, drain_out, 0)


@jax.jit
def kernel(cls_score, bbox_pred, labels, bbox_targets, inside_ws, outside_ws):
    iw_z, ow_z = _zero_call()

    lab2 = labels.reshape(N, 1)
    loss = pl.pallas_call(
        _loss_kernel,
        grid=(NBLK,),
        in_specs=[
            pl.BlockSpec((BLK, C), lambda i: (i, 0)),
            pl.BlockSpec((BLK, 1), lambda i: (i, 0)),
            pl.BlockSpec((BLK, BD), lambda i: (i, 0)),
            pl.BlockSpec((BLK, BD), lambda i: (i, 0)),
            pl.BlockSpec((BLK, BD), lambda i: (i, 0)),
            pl.BlockSpec((BLK, BD), lambda i: (i, 0)),
        ],
        out_specs=pl.BlockSpec((BLK, 1), lambda i: (i, 0)),
        out_shape=jax.ShapeDtypeStruct((N, 1), jnp.float32),
    )(cls_score, lab2, bbox_pred, bbox_targets, inside_ws, outside_ws)

    lab_ohem2d, gidxv, sbitsv = pl.pallas_call(
        _select_kernel,
        in_specs=[
            pl.BlockSpec((NBLK, BLK), lambda: (0, 0)),
            pl.BlockSpec((NBLK, BLK), lambda: (0, 0)),
        ],
        out_specs=[
            pl.BlockSpec((NBLK, BLK), lambda: (0, 0)),
            pl.BlockSpec((K, 1), lambda: (0, 0)),
            pl.BlockSpec((K, 1), lambda: (0, 0)),
        ],
        out_shape=[
            jax.ShapeDtypeStruct((NBLK, BLK), labels.dtype),
            jax.ShapeDtypeStruct((K, 1), jnp.int32),
            jax.ShapeDtypeStruct((K, 1), jnp.int32),
        ],
    )(loss.reshape(NBLK, BLK), labels.reshape(NBLK, BLK))

    grid_spec = pltpu.PrefetchScalarGridSpec(
        num_scalar_prefetch=1,
        grid=(1,),
        in_specs=[
            pl.BlockSpec((K, 1), lambda i, g: (0, 0)),
            pl.BlockSpec(memory_space=pl.ANY),
            pl.BlockSpec(memory_space=pl.ANY),
            pl.BlockSpec(memory_space=pl.ANY),
            pl.BlockSpec(memory_space=pl.ANY),
        ],
        out_specs=[
            pl.BlockSpec(memory_space=pl.ANY),
            pl.BlockSpec(memory_space=pl.ANY),
        ],
        scratch_shapes=[
            pltpu.VMEM((K, 8, BD), jnp.float32),
            pltpu.VMEM((K, 8, BD), jnp.float32),
            pltpu.SemaphoreType.DMA,
            pltpu.SemaphoreType.DMA,
        ],
    )
    iw_o, ow_o = pl.pallas_call(
        _scatter_kernel,
        grid_spec=grid_spec,
        out_shape=[
            jax.ShapeDtypeStruct((N, BD), jnp.float32),
            jax.ShapeDtypeStruct((N, BD), jnp.float32),
        ],
        input_output_aliases={4: 0, 5: 1},
    )(gidxv.reshape(K), sbitsv,
      inside_ws, outside_ws, iw_z, ow_z)

    return (lab_ohem2d.reshape(N), iw_o, ow_o)


# speedup vs baseline: 1.1966x; 1.0591x over previous
import functools

import jax
import jax.numpy as jnp
from jax import lax
from jax.experimental import pallas as pl
from jax.experimental.pallas import tpu as pltpu
from jax.experimental.pallas import tpu_sc as plsc

N = 20000
C = 81
BD = 324
K = 128
BLK = 1000
NBLK = N // BLK

SC_CORES = 2
SC_SUBCORES = 16
ZROWS = 200
ZCHUNKS = N // ZROWS
ZITER = -(-ZCHUNKS // SC_SUBCORES)


def _zero_body(iw_out, ow_out, zbuf):
    c = lax.axis_index("c")
    s = lax.axis_index("s")
    zero16 = jnp.zeros((16,), jnp.float32)

    def zrow(i, carry):
        for j in range(BD // 16):
            zbuf[i, pl.ds(j * 16, 16)] = zero16
        zbuf[i, pl.ds(BD - 16, 16)] = zero16
        return carry

    lax.fori_loop(0, ZROWS, zrow, 0)

    for k in range(ZITER):
        chunk = s + SC_SUBCORES * k

        @pl.when(jnp.logical_and(chunk < ZCHUNKS, c == 0))
        def _():
            pltpu.sync_copy(zbuf, iw_out.at[pl.ds(chunk * ZROWS, ZROWS), :])

        @pl.when(jnp.logical_and(chunk < ZCHUNKS, c == 1))
        def _():
            pltpu.sync_copy(zbuf, ow_out.at[pl.ds(chunk * ZROWS, ZROWS), :])


_zero_call = functools.partial(
    pl.kernel,
    mesh=plsc.VectorSubcoreMesh(core_axis_name="c", subcore_axis_name="s"),
    out_type=[
        jax.ShapeDtypeStruct((N, BD), jnp.float32),
        jax.ShapeDtypeStruct((N, BD), jnp.float32),
    ],
    scratch_types=[pltpu.VMEM((ZROWS, BD), jnp.float32)],
)(_zero_body)


def _loss_kernel(cls_ref, lab_ref, bp_ref, bt_ref, iw_ref, ow_ref, out_ref):
    x = cls_ref[...]
    lab = lab_ref[...]
    m = jnp.max(x, axis=1, keepdims=True)
    e = jnp.exp(x - m)
    s = jnp.sum(e, axis=1, keepdims=True)
    col = lax.broadcasted_iota(jnp.int32, (BLK, C), 1)
    xl = jnp.sum(jnp.where(col == lab, x, 0.0), axis=1, keepdims=True)
    loss_cls = jnp.log(s) + m - xl

    d = iw_ref[...] * (bp_ref[...] - bt_ref[...])
    a = jnp.abs(d)
    sign = (a < 1.0).astype(jnp.float32)
    in_loss = sign * 0.5 * d * d + (1.0 - sign) * (a - 0.5)
    loss_bbox = jnp.sum(ow_ref[...] * in_loss, axis=1, keepdims=True)

    out_ref[...] = loss_cls + loss_bbox


NG = N // 8


def _select_kernel(loss_ref, lab_ref, labout_ref, gidx_ref, sbits_ref):
    v = loss_ref[...]
    u = lax.bitcast_convert_type(v, jnp.int32)
    ks = jnp.where(u >= 0, u, u ^ jnp.int32(0x7FFFFFFF))

    c0 = jnp.sum((ks >= 0).astype(jnp.float32))
    t0 = jnp.where(c0 >= K, jnp.int32(0), jnp.int32(-(2**31)))

    def tbody(bi, t_acc):
        t = t_acc | lax.shift_left(jnp.int32(1), 30 - bi)
        cnt = jnp.sum((ks >= t).astype(jnp.float32))
        return jnp.where(cnt >= K, t, t_acc)

    tthr = lax.fori_loop(0, 31, tbody, t0)

    gt = ks > tthr
    eq = ks == tthr
    need = jnp.float32(K) - jnp.sum(gt.astype(jnp.float32))

    eqf = eq.astype(jnp.float32)
    lt_b = (lax.broadcasted_iota(jnp.int32, (BLK, BLK), 0)
            <= lax.broadcasted_iota(jnp.int32, (BLK, BLK), 1)
            ).astype(jnp.float32)
    incl = jnp.dot(eqf, lt_b, preferred_element_type=jnp.float32)
    rowtot = jnp.sum(eqf, axis=1, keepdims=True)
    ls_r = (lax.broadcasted_iota(jnp.int32, (NBLK, NBLK), 1)
            < lax.broadcasted_iota(jnp.int32, (NBLK, NBLK), 0)
            ).astype(jnp.float32)
    rowpre = jnp.dot(ls_r, rowtot, preferred_element_type=jnp.float32)
    rank = rowpre + incl - eqf
    keep = jnp.where(jnp.logical_or(gt, jnp.logical_and(eq, rank < need)),
                     1.0, 0.0)

    labout_ref[...] = jnp.where(keep > 0.0, lab_ref[...], -1)

    NQ = BLK // 8
    lane2 = lax.broadcasted_iota(jnp.int32, (NBLK, BLK), 1)
    w = lax.shift_left(jnp.int32(1), lane2 % 8).astype(jnp.float32)
    gsel = (lax.broadcasted_iota(jnp.int32, (BLK, NQ), 0) // 8
            == lax.broadcasted_iota(jnp.int32, (BLK, NQ), 1))
    bits = jnp.dot(keep * w, gsel.astype(jnp.float32),
                   preferred_element_type=jnp.float32)

    nzf = (bits > 0.0).astype(jnp.float32)
    cellf = (lax.broadcasted_iota(jnp.int32, (NBLK, NQ), 0) * NQ
             + lax.broadcasted_iota(jnp.int32, (NBLK, NQ), 1)
             ).astype(jnp.float32)
    lt_q = (lax.broadcasted_iota(jnp.int32, (NQ, NQ), 0)
            <= lax.broadcasted_iota(jnp.int32, (NQ, NQ), 1)
            ).astype(jnp.float32)
    incl2 = jnp.dot(nzf, lt_q, preferred_element_type=jnp.float32)
    rowtot2 = jnp.sum(nzf, axis=1, keepdims=True)
    rowpre2 = jnp.dot(ls_r, rowtot2, preferred_element_type=jnp.float32)
    srank = rowpre2 + incl2 - nzf
    cnt2 = jnp.sum(nzf)

    s_iota = lax.broadcasted_iota(jnp.int32, (NBLK, NQ, K), 2
                                  ).astype(jnp.float32)
    oh = jnp.where(jnp.logical_and(srank[:, :, None] == s_iota,
                                   nzf[:, :, None] > 0.0), 1.0, 0.0)
    gidx_l = jnp.sum(oh * cellf[:, :, None], axis=(0, 1))
    sbits_l = jnp.sum(oh * bits[:, :, None], axis=(0, 1))

    g1 = jnp.sum(jnp.where(srank == 0.0, nzf * cellf, 0.0))
    b1 = jnp.sum(jnp.where(srank == 0.0, nzf * bits, 0.0))
    sl = lax.broadcasted_iota(jnp.int32, (K,), 0).astype(jnp.float32)
    gidx_l = jnp.where(sl < cnt2, gidx_l, g1)
    sbits_l = jnp.where(sl < cnt2, sbits_l, b1)

    gidx_ref[...] = gidx_l.astype(jnp.int32).reshape(K, 1)
    sbits_ref[...] = sbits_l.astype(jnp.int32).reshape(K, 1)


def _scatter_kernel(g_ref, sbits_ref, iw_ref, ow_ref, iwz_ref, owz_ref,
                    iwo_ref, owo_ref, bufi, bufo, semg, semo):
    def fire(i, c):
        g = g_ref[i]
        pltpu.make_async_copy(
            iw_ref.at[pl.ds(g * 8, 8), :], bufi.at[i], semg).start()
        pltpu.make_async_copy(
            ow_ref.at[pl.ds(g * 8, 8), :], bufo.at[i], semg).start()
        return c

    lax.fori_loop(0, K, fire, 0)

    def drain_in(i, c):
        g = g_ref[i]
        pltpu.make_async_copy(
            iw_ref.at[pl.ds(g * 8, 8), :], bufi.at[i], semg).wait()
        pltpu.make_async_copy(
            ow_ref.at[pl.ds(g * 8, 8), :], bufo.at[i], semg).wait()
        return c

    lax.fori_loop(0, K, drain_in, 0)

    sb3 = sbits_ref[...].reshape(K, 1, 1)
    sub3 = lax.broadcasted_iota(jnp.int32, (1, 8, 1), 1)
    rowm = (lax.shift_right_logical(sb3, sub3) & 1) > 0
    bufi[...] = jnp.where(rowm, bufi[...], 0.0)
    bufo[...] = jnp.where(rowm, bufo[...], 0.0)

    def fire_out(i, c):
        g = g_ref[i]
        pltpu.make_async_copy(
            bufi.at[i], iwo_ref.at[pl.ds(g * 8, 8), :], semo).start()
        pltpu.make_async_copy(
            bufo.at[i], owo_ref.at[pl.ds(g * 8, 8), :], semo).start()
        return c

    lax.fori_loop(0, K, fire_out, 0)

    def drain_out(i, c):
        g = g_ref[i]
        pltpu.make_async_copy(
            bufi.at[i], iwo_ref.at[pl.ds(g * 8, 8), :], semo).wait()
        pltpu.make_async_copy(
            bufo.at[i], owo_ref.at[pl.ds(g * 8, 8), :], semo).wait()
        return c

    lax.fori_loop(0, K, drain_out, 0)


@jax.jit
def kernel(cls_score, bbox_pred, labels, bbox_targets, inside_ws, outside_ws):
    iw_z, ow_z = _zero_call()

    lab2 = labels.reshape(N, 1)
    loss = pl.pallas_call(
        _loss_kernel,
        grid=(NBLK,),
        in_specs=[
            pl.BlockSpec((BLK, C), lambda i: (i, 0)),
            pl.BlockSpec((BLK, 1), lambda i: (i, 0)),
            pl.BlockSpec((BLK, BD), lambda i: (i, 0)),
            pl.BlockSpec((BLK, BD), lambda i: (i, 0)),
            pl.BlockSpec((BLK, BD), lambda i: (i, 0)),
            pl.BlockSpec((BLK, BD), lambda i: (i, 0)),
        ],
        out_specs=pl.BlockSpec((BLK, 1), lambda i: (i, 0)),
        out_shape=jax.ShapeDtypeStruct((N, 1), jnp.float32),
    )(cls_score, lab2, bbox_pred, bbox_targets, inside_ws, outside_ws)

    lab_ohem2d, gidxv, sbitsv = pl.pallas_call(
        _select_kernel,
        in_specs=[
            pl.BlockSpec((NBLK, BLK), lambda: (0, 0)),
            pl.BlockSpec((NBLK, BLK), lambda: (0, 0)),
        ],
        out_specs=[
            pl.BlockSpec((NBLK, BLK), lambda: (0, 0)),
            pl.BlockSpec((K, 1), lambda: (0, 0)),
            pl.BlockSpec((K, 1), lambda: (0, 0)),
        ],
        out_shape=[
            jax.ShapeDtypeStruct((NBLK, BLK), labels.dtype),
            jax.ShapeDtypeStruct((K, 1), jnp.int32),
            jax.ShapeDtypeStruct((K, 1), jnp.int32),
        ],
    )(loss.reshape(NBLK, BLK), labels.reshape(NBLK, BLK))

    grid_spec = pltpu.PrefetchScalarGridSpec(
        num_scalar_prefetch=1,
        grid=(1,),
        in_specs=[
            pl.BlockSpec((K, 1), lambda i, g: (0, 0)),
            pl.BlockSpec(memory_space=pl.ANY),
            pl.BlockSpec(memory_space=pl.ANY),
            pl.BlockSpec(memory_space=pl.ANY),
            pl.BlockSpec(memory_space=pl.ANY),
        ],
        out_specs=[
            pl.BlockSpec(memory_space=pl.ANY),
            pl.BlockSpec(memory_space=pl.ANY),
        ],
        scratch_shapes=[
            pltpu.VMEM((K, 8, BD), jnp.float32),
            pltpu.VMEM((K, 8, BD), jnp.float32),
            pltpu.SemaphoreType.DMA,
            pltpu.SemaphoreType.DMA,
        ],
    )
    iw_o, ow_o = pl.pallas_call(
        _scatter_kernel,
        grid_spec=grid_spec,
        out_shape=[
            jax.ShapeDtypeStruct((N, BD), jnp.float32),
            jax.ShapeDtypeStruct((N, BD), jnp.float32),
        ],
        input_output_aliases={4: 0, 5: 1},
    )(gidxv.reshape(K), sbitsv,
      inside_ws, outside_ws, iw_z, ow_z)

    return (lab_ohem2d.reshape(N), iw_o, ow_o)


# speedup vs baseline: 1.1985x; 1.0015x over previous
import functools

import jax
import jax.numpy as jnp
from jax import lax
from jax.experimental import pallas as pl
from jax.experimental.pallas import tpu as pltpu
from jax.experimental.pallas import tpu_sc as plsc

N = 20000
C = 81
BD = 324
K = 128
BLK = 2000
NBLK = N // BLK

SC_CORES = 2
SC_SUBCORES = 16
ZROWS = 200
ZCHUNKS = N // ZROWS
ZITER = -(-ZCHUNKS // SC_SUBCORES)


def _zero_body(iw_out, ow_out, zbuf):
    c = lax.axis_index("c")
    s = lax.axis_index("s")
    zero16 = jnp.zeros((16,), jnp.float32)

    def zrow(i, carry):
        for j in range(BD // 16):
            zbuf[i, pl.ds(j * 16, 16)] = zero16
        zbuf[i, pl.ds(BD - 16, 16)] = zero16
        return carry

    lax.fori_loop(0, ZROWS, zrow, 0)

    for k in range(ZITER):
        chunk = s + SC_SUBCORES * k

        @pl.when(jnp.logical_and(chunk < ZCHUNKS, c == 0))
        def _():
            pltpu.sync_copy(zbuf, iw_out.at[pl.ds(chunk * ZROWS, ZROWS), :])

        @pl.when(jnp.logical_and(chunk < ZCHUNKS, c == 1))
        def _():
            pltpu.sync_copy(zbuf, ow_out.at[pl.ds(chunk * ZROWS, ZROWS), :])


_zero_call = functools.partial(
    pl.kernel,
    mesh=plsc.VectorSubcoreMesh(core_axis_name="c", subcore_axis_name="s"),
    out_type=[
        jax.ShapeDtypeStruct((N, BD), jnp.float32),
        jax.ShapeDtypeStruct((N, BD), jnp.float32),
    ],
    scratch_types=[pltpu.VMEM((ZROWS, BD), jnp.float32)],
)(_zero_body)


def _loss_kernel(cls_ref, lab_ref, bp_ref, bt_ref, iw_ref, ow_ref, out_ref):
    x = cls_ref[...]
    lab = lab_ref[...]
    m = jnp.max(x, axis=1, keepdims=True)
    e = jnp.exp(x - m)
    s = jnp.sum(e, axis=1, keepdims=True)
    col = lax.broadcasted_iota(jnp.int32, (BLK, C), 1)
    xl = jnp.sum(jnp.where(col == lab, x, 0.0), axis=1, keepdims=True)
    loss_cls = jnp.log(s) + m - xl

    d = iw_ref[...] * (bp_ref[...] - bt_ref[...])
    a = jnp.abs(d)
    sign = (a < 1.0).astype(jnp.float32)
    in_loss = sign * 0.5 * d * d + (1.0 - sign) * (a - 0.5)
    loss_bbox = jnp.sum(ow_ref[...] * in_loss, axis=1, keepdims=True)

    out_ref[...] = loss_cls + loss_bbox


NG = N // 8


def _select_kernel(loss_ref, lab_ref, labout_ref, gidx_ref, sbits_ref):
    v = loss_ref[...]
    u = lax.bitcast_convert_type(v, jnp.int32)
    ks = jnp.where(u >= 0, u, u ^ jnp.int32(0x7FFFFFFF))

    c0 = jnp.sum((ks >= 0).astype(jnp.float32))
    t0 = jnp.where(c0 >= K, jnp.int32(0), jnp.int32(-(2**31)))

    def tbody(bi, t_acc):
        t = t_acc | lax.shift_left(jnp.int32(1), 30 - bi)
        cnt = jnp.sum((ks >= t).astype(jnp.float32))
        return jnp.where(cnt >= K, t, t_acc)

    tthr = lax.fori_loop(0, 31, tbody, t0)

    gt = ks > tthr
    eq = ks == tthr
    need = jnp.float32(K) - jnp.sum(gt.astype(jnp.float32))

    eqf = eq.astype(jnp.float32)
    lt_b = (lax.broadcasted_iota(jnp.int32, (BLK, BLK), 0)
            <= lax.broadcasted_iota(jnp.int32, (BLK, BLK), 1)
            ).astype(jnp.float32)
    incl = jnp.dot(eqf, lt_b, preferred_element_type=jnp.float32)
    rowtot = jnp.sum(eqf, axis=1, keepdims=True)
    ls_r = (lax.broadcasted_iota(jnp.int32, (NBLK, NBLK), 1)
            < lax.broadcasted_iota(jnp.int32, (NBLK, NBLK), 0)
            ).astype(jnp.float32)
    rowpre = jnp.dot(ls_r, rowtot, preferred_element_type=jnp.float32)
    rank = rowpre + incl - eqf
    keep = jnp.where(jnp.logical_or(gt, jnp.logical_and(eq, rank < need)),
                     1.0, 0.0)

    labout_ref[...] = jnp.where(keep > 0.0, lab_ref[...], -1)

    NQ = BLK // 8
    lane2 = lax.broadcasted_iota(jnp.int32, (NBLK, BLK), 1)
    w = lax.shift_left(jnp.int32(1), lane2 % 8).astype(jnp.float32)
    gsel = (lax.broadcasted_iota(jnp.int32, (BLK, NQ), 0) // 8
            == lax.broadcasted_iota(jnp.int32, (BLK, NQ), 1))
    bits = jnp.dot(keep * w, gsel.astype(jnp.float32),
                   preferred_element_type=jnp.float32)

    nzf = (bits > 0.0).astype(jnp.float32)
    cellf = (lax.broadcasted_iota(jnp.int32, (NBLK, NQ), 0) * NQ
             + lax.broadcasted_iota(jnp.int32, (NBLK, NQ), 1)
             ).astype(jnp.float32)
    lt_q = (lax.broadcasted_iota(jnp.int32, (NQ, NQ), 0)
            <= lax.broadcasted_iota(jnp.int32, (NQ, NQ), 1)
            ).astype(jnp.float32)
    incl2 = jnp.dot(nzf, lt_q, preferred_element_type=jnp.float32)
    rowtot2 = jnp.sum(nzf, axis=1, keepdims=True)
    rowpre2 = jnp.dot(ls_r, rowtot2, preferred_element_type=jnp.float32)
    srank = rowpre2 + incl2 - nzf
    cnt2 = jnp.sum(nzf)

    s_iota = lax.broadcasted_iota(jnp.int32, (NBLK, NQ, K), 2
                                  ).astype(jnp.float32)
    oh = jnp.where(jnp.logical_and(srank[:, :, None] == s_iota,
                                   nzf[:, :, None] > 0.0), 1.0, 0.0)
    gidx_l = jnp.sum(oh * cellf[:, :, None], axis=(0, 1))
    sbits_l = jnp.sum(oh * bits[:, :, None], axis=(0, 1))

    g1 = jnp.sum(jnp.where(srank == 0.0, nzf * cellf, 0.0))
    b1 = jnp.sum(jnp.where(srank == 0.0, nzf * bits, 0.0))
    sl = lax.broadcasted_iota(jnp.int32, (K,), 0).astype(jnp.float32)
    gidx_l = jnp.where(sl < cnt2, gidx_l, g1)
    sbits_l = jnp.where(sl < cnt2, sbits_l, b1)

    gidx_ref[...] = gidx_l.astype(jnp.int32).reshape(K, 1)
    sbits_ref[...] = sbits_l.astype(jnp.int32).reshape(K, 1)


def _scatter_kernel(g_ref, sbits_ref, iw_ref, ow_ref, iwz_ref, owz_ref,
                    iwo_ref, owo_ref, bufi, bufo, semg, semo):
    def fire(i, c):
        g = g_ref[i]
        pltpu.make_async_copy(
            iw_ref.at[pl.ds(g * 8, 8), :], bufi.at[i], semg).start()
        pltpu.make_async_copy(
            ow_ref.at[pl.ds(g * 8, 8), :], bufo.at[i], semg).start()
        return c

    lax.fori_loop(0, K, fire, 0)

    def drain_in(i, c):
        g = g_ref[i]
        pltpu.make_async_copy(
            iw_ref.at[pl.ds(g * 8, 8), :], bufi.at[i], semg).wait()
        pltpu.make_async_copy(
            ow_ref.at[pl.ds(g * 8, 8), :], bufo.at[i], semg).wait()
        return c

    lax.fori_loop(0, K, drain_in, 0)

    sb3 = sbits_ref[...].reshape(K, 1, 1)
    sub3 = lax.broadcasted_iota(jnp.int32, (1, 8, 1), 1)
    rowm = (lax.shift_right_logical(sb3, sub3) & 1) > 0
    bufi[...] = jnp.where(rowm, bufi[...], 0.0)
    bufo[...] = jnp.where(rowm, bufo[...], 0.0)

    def fire_out(i, c):
        g = g_ref[i]
        pltpu.make_async_copy(
            bufi.at[i], iwo_ref.at[pl.ds(g * 8, 8), :], semo).start()
        pltpu.make_async_copy(
            bufo.at[i], owo_ref.at[pl.ds(g * 8, 8), :], semo).start()
        return c

    lax.fori_loop(0, K, fire_out, 0)

    def drain_out(i, c):
        g = g_ref[i]
        pltpu.make_async_copy(
            bufi.at[i], iwo_ref.at[pl.ds(g * 8, 8), :], semo).wait()
        pltpu.make_async_copy(
            bufo.at[i], owo_ref.at[pl.ds(g * 8, 8), :], semo).wait()
        return c

    lax.fori_loop(0, K, drain_out, 0)


@jax.jit
def kernel(cls_score, bbox_pred, labels, bbox_targets, inside_ws, outside_ws):
    iw_z, ow_z = _zero_call()

    lab2 = labels.reshape(N, 1)
    loss = pl.pallas_call(
        _loss_kernel,
        grid=(NBLK,),
        in_specs=[
            pl.BlockSpec((BLK, C), lambda i: (i, 0)),
            pl.BlockSpec((BLK, 1), lambda i: (i, 0)),
            pl.BlockSpec((BLK, BD), lambda i: (i, 0)),
            pl.BlockSpec((BLK, BD), lambda i: (i, 0)),
            pl.BlockSpec((BLK, BD), lambda i: (i, 0)),
            pl.BlockSpec((BLK, BD), lambda i: (i, 0)),
        ],
        out_specs=pl.BlockSpec((BLK, 1), lambda i: (i, 0)),
        out_shape=jax.ShapeDtypeStruct((N, 1), jnp.float32),
    )(cls_score, lab2, bbox_pred, bbox_targets, inside_ws, outside_ws)

    lab_ohem2d, gidxv, sbitsv = pl.pallas_call(
        _select_kernel,
        in_specs=[
            pl.BlockSpec((NBLK, BLK), lambda: (0, 0)),
            pl.BlockSpec((NBLK, BLK), lambda: (0, 0)),
        ],
        out_specs=[
            pl.BlockSpec((NBLK, BLK), lambda: (0, 0)),
            pl.BlockSpec((K, 1), lambda: (0, 0)),
            pl.BlockSpec((K, 1), lambda: (0, 0)),
        ],
        out_shape=[
            jax.ShapeDtypeStruct((NBLK, BLK), labels.dtype),
            jax.ShapeDtypeStruct((K, 1), jnp.int32),
            jax.ShapeDtypeStruct((K, 1), jnp.int32),
        ],
    )(loss.reshape(NBLK, BLK), labels.reshape(NBLK, BLK))

    grid_spec = pltpu.PrefetchScalarGridSpec(
        num_scalar_prefetch=1,
        grid=(1,),
        in_specs=[
            pl.BlockSpec((K, 1), lambda i, g: (0, 0)),
            pl.BlockSpec(memory_space=pl.ANY),
            pl.BlockSpec(memory_space=pl.ANY),
            pl.BlockSpec(memory_space=pl.ANY),
            pl.BlockSpec(memory_space=pl.ANY),
        ],
        out_specs=[
            pl.BlockSpec(memory_space=pl.ANY),
            pl.BlockSpec(memory_space=pl.ANY),
        ],
        scratch_shapes=[
            pltpu.VMEM((K, 8, BD), jnp.float32),
            pltpu.VMEM((K, 8, BD), jnp.float32),
            pltpu.SemaphoreType.DMA,
            pltpu.SemaphoreType.DMA,
        ],
    )
    iw_o, ow_o = pl.pallas_call(
        _scatter_kernel,
        grid_spec=grid_spec,
        out_shape=[
            jax.ShapeDtypeStruct((N, BD), jnp.float32),
            jax.ShapeDtypeStruct((N, BD), jnp.float32),
        ],
        input_output_aliases={4: 0, 5: 1},
    )(gidxv.reshape(K), sbitsv,
      inside_ws, outside_ws, iw_z, ow_z)

    return (lab_ohem2d.reshape(N), iw_o, ow_o)
